# Initial kernel scaffold; baseline (speedup 1.0000x reference)
import jax
import jax.numpy as jnp
from jax.experimental import pallas as pl


def kernel(X, A, W_gat, a_gat, W1, b1, W2, b2):
    raise NotImplementedError("write your pallas kernel here")


# speedup vs baseline: 1.6396x; 1.6396x over previous
import functools

import jax
import jax.numpy as jnp
from jax.experimental import pallas as pl
from jax.experimental.pallas import tpu as pltpu


def _proj_kernel(x_ref, w_ref, a1_ref, a2_ref, h_ref, f1_ref, f2_ref):
    h = jnp.dot(x_ref[0], w_ref[...], preferred_element_type=jnp.float32)
    h_ref[0] = h
    f1_ref[0] = jnp.dot(h, a1_ref[...], preferred_element_type=jnp.float32)
    f2_ref[0] = jnp.dot(h, a2_ref[...], preferred_element_type=jnp.float32)


def _gat_kernel(a_ref, h_ref, x_ref, f1_ref, f2t_ref, acc_ref):
    i = pl.program_id(1)
    e = f1_ref[0] + f2t_ref[0]
    e = jnp.where(e >= 0.0, e, 0.2 * e)
    logits = jnp.where(a_ref[0] > 0.0, e, jnp.float32(-9e15))
    m = jnp.max(logits, axis=1, keepdims=True)
    p = jnp.exp(logits - m)
    s = jnp.sum(p, axis=1, keepdims=True)
    att = p / s
    hp = jnp.dot(att, h_ref[0], preferred_element_type=jnp.float32)
    contrib = x_ref[0] + hp + 2.0 * jnp.maximum(hp, 0.0)
    part = jnp.sum(contrib, axis=0, keepdims=True)

    @pl.when(i == 0)
    def _():
        acc_ref[...] = jnp.zeros_like(acc_ref)

    acc_ref[...] += part[None]


def _mlp_kernel(inv_pool, acc_ref, w1_ref, b1_ref, w2_ref, b2_ref, out_ref):
    xm = acc_ref[...] * inv_pool
    hmid = jnp.dot(xm, w1_ref[...], preferred_element_type=jnp.float32) + b1_ref[...]
    hmid = jnp.maximum(hmid, 0.0)
    out_ref[...] = jnp.dot(hmid, w2_ref[...], preferred_element_type=jnp.float32) + b2_ref[...]


def kernel(X, A, W_gat, a_gat, W1, b1, W2, b2):
    B, N, F = X.shape
    H = W1.shape[1]
    BLK = 256
    a1 = a_gat[:F]
    a2 = a_gat[F:]

    h, f1, f2 = pl.pallas_call(
        _proj_kernel,
        grid=(B,),
        in_specs=[
            pl.BlockSpec((1, N, F), lambda b: (b, 0, 0)),
            pl.BlockSpec((F, F), lambda b: (0, 0)),
            pl.BlockSpec((F, 1), lambda b: (0, 0)),
            pl.BlockSpec((F, 1), lambda b: (0, 0)),
        ],
        out_specs=[
            pl.BlockSpec((1, N, F), lambda b: (b, 0, 0)),
            pl.BlockSpec((1, N, 1), lambda b: (b, 0, 0)),
            pl.BlockSpec((1, N, 1), lambda b: (b, 0, 0)),
        ],
        out_shape=[
            jax.ShapeDtypeStruct((B, N, F), jnp.float32),
            jax.ShapeDtypeStruct((B, N, 1), jnp.float32),
            jax.ShapeDtypeStruct((B, N, 1), jnp.float32),
        ],
    )(X, W_gat, a1, a2)

    f2t = jnp.reshape(f2, (B, 1, N))

    acc = pl.pallas_call(
        _gat_kernel,
        grid=(B, N // BLK),
        in_specs=[
            pl.BlockSpec((1, BLK, N), lambda b, i: (b, i, 0)),
            pl.BlockSpec((1, N, F), lambda b, i: (b, 0, 0)),
            pl.BlockSpec((1, BLK, F), lambda b, i: (b, i, 0)),
            pl.BlockSpec((1, BLK, 1), lambda b, i: (b, i, 0)),
            pl.BlockSpec((1, 1, N), lambda b, i: (b, 0, 0)),
        ],
        out_specs=pl.BlockSpec((1, 1, F), lambda b, i: (b, 0, 0)),
        out_shape=jax.ShapeDtypeStruct((B, 1, F), jnp.float32),
        compiler_params=pltpu.CompilerParams(
            dimension_semantics=("parallel", "arbitrary"),
        ),
    )(A, h, X, f1, f2t)

    out = pl.pallas_call(
        functools.partial(_mlp_kernel, 1.0 / (4.0 * N)),
        in_specs=[
            pl.BlockSpec((B, F), lambda: (0, 0)),
            pl.BlockSpec((F, H), lambda: (0, 0)),
            pl.BlockSpec((1, H), lambda: (0, 0)),
            pl.BlockSpec((H, 1), lambda: (0, 0)),
            pl.BlockSpec((1, 1), lambda: (0, 0)),
        ],
        out_specs=pl.BlockSpec((B, 1), lambda: (0, 0)),
        out_shape=jax.ShapeDtypeStruct((B, 1), jnp.float32),
    )(acc.reshape(B, F), W1, b1.reshape(1, H), W2, b2.reshape(1, 1))

    return out


# speedup vs baseline: 1.7725x; 1.0810x over previous
import functools

import jax
import jax.numpy as jnp
from jax.experimental import pallas as pl
from jax.experimental.pallas import tpu as pltpu


def _proj_kernel(x_ref, w_ref, a1_ref, a2_ref, h_ref, f1_ref, f2_ref):
    h = jnp.dot(x_ref[0], w_ref[...], preferred_element_type=jnp.float32)
    h_ref[0] = h
    f1_ref[0] = jnp.dot(h, a1_ref[...], preferred_element_type=jnp.float32)
    f2_ref[0] = jnp.dot(h, a2_ref[...], preferred_element_type=jnp.float32)


def _gat_kernel(a_ref, h_ref, x_ref, f1_ref, f2t_ref, acc_ref):
    i = pl.program_id(1)
    e = f1_ref[0] + f2t_ref[0]
    e = jnp.where(e >= 0.0, e, 0.2 * e)
    logits = jnp.where(a_ref[0] > 0.0, e, jnp.float32(-9e15))
    m = jnp.max(logits, axis=1, keepdims=True)
    p = jnp.exp(logits - m)
    s = jnp.sum(p, axis=1, keepdims=True)
    hp = jnp.dot(p, h_ref[0], preferred_element_type=jnp.float32) / s
    contrib = x_ref[0] + hp + 2.0 * jnp.maximum(hp, 0.0)
    part = jnp.sum(contrib, axis=0, keepdims=True)

    @pl.when(i == 0)
    def _():
        acc_ref[...] = jnp.zeros_like(acc_ref)

    acc_ref[...] += part[None]


def _mlp_kernel(inv_pool, acc_ref, w1_ref, b1_ref, w2_ref, b2_ref, out_ref):
    xm = acc_ref[...] * inv_pool
    hmid = jnp.dot(xm, w1_ref[...], preferred_element_type=jnp.float32) + b1_ref[...]
    hmid = jnp.maximum(hmid, 0.0)
    out_ref[...] = jnp.dot(hmid, w2_ref[...], preferred_element_type=jnp.float32) + b2_ref[...]


def kernel(X, A, W_gat, a_gat, W1, b1, W2, b2):
    B, N, F = X.shape
    H = W1.shape[1]
    BLK = 512
    a1 = a_gat[:F]
    a2 = a_gat[F:]

    h, f1, f2 = pl.pallas_call(
        _proj_kernel,
        grid=(B,),
        in_specs=[
            pl.BlockSpec((1, N, F), lambda b: (b, 0, 0)),
            pl.BlockSpec((F, F), lambda b: (0, 0)),
            pl.BlockSpec((F, 1), lambda b: (0, 0)),
            pl.BlockSpec((F, 1), lambda b: (0, 0)),
        ],
        out_specs=[
            pl.BlockSpec((1, N, F), lambda b: (b, 0, 0)),
            pl.BlockSpec((1, N, 1), lambda b: (b, 0, 0)),
            pl.BlockSpec((1, N, 1), lambda b: (b, 0, 0)),
        ],
        out_shape=[
            jax.ShapeDtypeStruct((B, N, F), jnp.float32),
            jax.ShapeDtypeStruct((B, N, 1), jnp.float32),
            jax.ShapeDtypeStruct((B, N, 1), jnp.float32),
        ],
    )(X, W_gat, a1, a2)

    f2t = jnp.reshape(f2, (B, 1, N))

    acc = pl.pallas_call(
        _gat_kernel,
        grid=(B, N // BLK),
        in_specs=[
            pl.BlockSpec((1, BLK, N), lambda b, i: (b, i, 0)),
            pl.BlockSpec((1, N, F), lambda b, i: (b, 0, 0)),
            pl.BlockSpec((1, BLK, F), lambda b, i: (b, i, 0)),
            pl.BlockSpec((1, BLK, 1), lambda b, i: (b, i, 0)),
            pl.BlockSpec((1, 1, N), lambda b, i: (b, 0, 0)),
        ],
        out_specs=pl.BlockSpec((1, 1, F), lambda b, i: (b, 0, 0)),
        out_shape=jax.ShapeDtypeStruct((B, 1, F), jnp.float32),
        compiler_params=pltpu.CompilerParams(
            dimension_semantics=("parallel", "arbitrary"),
        ),
    )(A, h, X, f1, f2t)

    out = pl.pallas_call(
        functools.partial(_mlp_kernel, 1.0 / (4.0 * N)),
        in_specs=[
            pl.BlockSpec((B, F), lambda: (0, 0)),
            pl.BlockSpec((F, H), lambda: (0, 0)),
            pl.BlockSpec((1, H), lambda: (0, 0)),
            pl.BlockSpec((H, 1), lambda: (0, 0)),
            pl.BlockSpec((1, 1), lambda: (0, 0)),
        ],
        out_specs=pl.BlockSpec((B, 1), lambda: (0, 0)),
        out_shape=jax.ShapeDtypeStruct((B, 1), jnp.float32),
    )(acc.reshape(B, F), W1, b1.reshape(1, H), W2, b2.reshape(1, 1))

    return out


# speedup vs baseline: 2.0368x; 1.1491x over previous
import functools

import jax
import jax.numpy as jnp
from jax.experimental import pallas as pl
from jax.experimental.pallas import tpu as pltpu


_LOG2E = 1.4426950408889634


def _proj_kernel(x_ref, w_ref, a1_ref, a2_ref, h_ref, f1_ref, f2_ref, f2max_ref):
    h = jnp.dot(x_ref[0], w_ref[...], preferred_element_type=jnp.float32)
    h_ref[0] = h
    f1 = jnp.dot(h, a1_ref[...], preferred_element_type=jnp.float32) * _LOG2E
    f2 = jnp.dot(h, a2_ref[...], preferred_element_type=jnp.float32) * _LOG2E
    f1_ref[0] = f1
    f2_ref[0] = f2
    f2max_ref[0] = jnp.max(f2, axis=0, keepdims=True)


def _gat_kernel(a_ref, h_ref, x_ref, f1_ref, f2t_ref, f2max_ref, acc_ref):
    i = pl.program_id(1)
    f1 = f1_ref[0]
    tmax = f1 + f2max_ref[0]
    mrow = jnp.maximum(tmax, 0.2 * tmax)
    t = f1 + f2t_ref[0]
    e = jnp.maximum(t, 0.2 * t)
    q = jnp.exp2(e - mrow)
    p = jnp.where(a_ref[0] > 0.0, q, 0.0)
    s = jnp.sum(p, axis=1, keepdims=True)
    s = jnp.maximum(s, jnp.float32(1e-30))
    hp = jnp.dot(p, h_ref[0], preferred_element_type=jnp.float32) / s
    contrib = x_ref[0] + hp + 2.0 * jnp.maximum(hp, 0.0)
    part = jnp.sum(contrib, axis=0, keepdims=True)

    @pl.when(i == 0)
    def _():
        acc_ref[...] = jnp.zeros_like(acc_ref)

    acc_ref[...] += part[None]


def _mlp_kernel(inv_pool, acc_ref, w1_ref, b1_ref, w2_ref, b2_ref, out_ref):
    xm = acc_ref[...] * inv_pool
    hmid = jnp.dot(xm, w1_ref[...], preferred_element_type=jnp.float32) + b1_ref[...]
    hmid = jnp.maximum(hmid, 0.0)
    out_ref[...] = jnp.dot(hmid, w2_ref[...], preferred_element_type=jnp.float32) + b2_ref[...]


def kernel(X, A, W_gat, a_gat, W1, b1, W2, b2):
    B, N, F = X.shape
    H = W1.shape[1]
    BLK = 512
    a1 = a_gat[:F]
    a2 = a_gat[F:]

    h, f1, f2, f2max = pl.pallas_call(
        _proj_kernel,
        grid=(B,),
        in_specs=[
            pl.BlockSpec((1, N, F), lambda b: (b, 0, 0)),
            pl.BlockSpec((F, F), lambda b: (0, 0)),
            pl.BlockSpec((F, 1), lambda b: (0, 0)),
            pl.BlockSpec((F, 1), lambda b: (0, 0)),
        ],
        out_specs=[
            pl.BlockSpec((1, N, F), lambda b: (b, 0, 0)),
            pl.BlockSpec((1, N, 1), lambda b: (b, 0, 0)),
            pl.BlockSpec((1, N, 1), lambda b: (b, 0, 0)),
            pl.BlockSpec((1, 1, 1), lambda b: (b, 0, 0)),
        ],
        out_shape=[
            jax.ShapeDtypeStruct((B, N, F), jnp.float32),
            jax.ShapeDtypeStruct((B, N, 1), jnp.float32),
            jax.ShapeDtypeStruct((B, N, 1), jnp.float32),
            jax.ShapeDtypeStruct((B, 1, 1), jnp.float32),
        ],
    )(X, W_gat, a1, a2)

    f2t = jnp.reshape(f2, (B, 1, N))

    acc = pl.pallas_call(
        _gat_kernel,
        grid=(B, N // BLK),
        in_specs=[
            pl.BlockSpec((1, BLK, N), lambda b, i: (b, i, 0)),
            pl.BlockSpec((1, N, F), lambda b, i: (b, 0, 0)),
            pl.BlockSpec((1, BLK, F), lambda b, i: (b, i, 0)),
            pl.BlockSpec((1, BLK, 1), lambda b, i: (b, i, 0)),
            pl.BlockSpec((1, 1, N), lambda b, i: (b, 0, 0)),
            pl.BlockSpec((1, 1, 1), lambda b, i: (b, 0, 0)),
        ],
        out_specs=pl.BlockSpec((1, 1, F), lambda b, i: (b, 0, 0)),
        out_shape=jax.ShapeDtypeStruct((B, 1, F), jnp.float32),
        compiler_params=pltpu.CompilerParams(
            dimension_semantics=("parallel", "arbitrary"),
        ),
    )(A, h, X, f1, f2t, f2max)

    out = pl.pallas_call(
        functools.partial(_mlp_kernel, 1.0 / (4.0 * N)),
        in_specs=[
            pl.BlockSpec((B, F), lambda: (0, 0)),
            pl.BlockSpec((F, H), lambda: (0, 0)),
            pl.BlockSpec((1, H), lambda: (0, 0)),
            pl.BlockSpec((H, 1), lambda: (0, 0)),
            pl.BlockSpec((1, 1), lambda: (0, 0)),
        ],
        out_specs=pl.BlockSpec((B, 1), lambda: (0, 0)),
        out_shape=jax.ShapeDtypeStruct((B, 1), jnp.float32),
    )(acc.reshape(B, F), W1, b1.reshape(1, H), W2, b2.reshape(1, 1))

    return out
